# baseline (speedup 1.0000x reference)
import functools

import jax
import jax.numpy as jnp
from jax import lax
from jax.experimental import pallas as pl
from jax.experimental.pallas import tpu as pltpu
from jax.experimental.pallas import tpu_sc as plsc


def kernel(q, k_compressed, v_compressed):
    B, S, H, D = q.shape
    CBS = k_compressed.shape[3]
    HC = H * CBS
    DT, DI = D // 8, 8
    ST, SI = S // 128, 128

    ksrc = k_compressed.reshape(HC, D)
    vsrc = v_compressed.reshape(HC, D)

    info = plsc.get_sparse_core_info()
    NC = info.num_cores
    NW = NC * info.num_subcores
    BPW = HC // NW
    DTP = 2
    NP = DT // DTP
    JOBS = BPW * NP
    RING = 3

    mesh = plsc.VectorSubcoreMesh(core_axis_name="c", subcore_axis_name="s")

    @functools.partial(
        pl.kernel,
        mesh=mesh,
        compiler_params=pltpu.CompilerParams(
            use_tc_tiling_on_sc=False, needs_layout_passes=False
        ),
        out_type=[jax.ShapeDtypeStruct((HC, DT, ST, DI, SI), jnp.float32)],
        scratch_types=[
            pltpu.VMEM((HC, D), jnp.float32),
            pltpu.VMEM((RING, DTP, ST, DI, SI), jnp.float32),
            pltpu.SemaphoreType.DMA,
        ],
    )
    def sc_write(ks_hbm, ko_hbm, ksv, buf, sem):
        wid = lax.axis_index("s") * NC + lax.axis_index("c")
        base = wid * BPW
        pltpu.sync_copy(ks_hbm, ksv)

        def job(j, carry):
            hc = base + j // NP
            dtp = j % NP
            slot = j % RING

            @pl.when(j >= RING)
            def _():
                pltpu.make_async_copy(
                    ko_hbm.at[0, pl.ds(0, DTP)], buf.at[0], sem
                ).wait()

            hc_i = jnp.full((16,), hc, jnp.int32)
            splats = [
                plsc.load_gather(
                    ksv,
                    [hc_i, jnp.full((16,), dtp * DTP * DI + di2, jnp.int32)],
                )
                for di2 in range(DTP * DI)
            ]

            def fill(st, c):
                for di2 in range(DTP * DI):
                    for i in range(SI // 16):
                        buf[slot, di2 // DI, st, di2 % DI, pl.ds(i * 16, 16)] = (
                            splats[di2]
                        )
                return c

            lax.fori_loop(0, ST, fill, 0)
            pltpu.async_copy(
                buf.at[slot], ko_hbm.at[hc, pl.ds(dtp * DTP, DTP)], sem
            )
            return carry

        lax.fori_loop(0, JOBS, job, 0)
        for _ in range(RING):
            pltpu.make_async_copy(ko_hbm.at[0, pl.ds(0, DTP)], buf.at[0], sem).wait()

    (k_out,) = sc_write(ksrc)

    GC = 16
    grid = (HC // GC,)

    def tc_body(vs_ref, vo_ref):
        vo_ref[...] = jnp.broadcast_to(vs_ref[...][:, :, None], (GC, D, S))

    v_out = pl.pallas_call(
        tc_body,
        grid=grid,
        in_specs=[pl.BlockSpec((GC, D), lambda i: (i, 0))],
        out_specs=[pl.BlockSpec((GC, D, S), lambda i: (i, 0, 0))],
        out_shape=[jax.ShapeDtypeStruct((HC, D, S), jnp.float32)],
    )(vsrc)[0]

    k_sel = jnp.transpose(
        k_out.reshape(H, CBS, DT, ST, DI, SI), (3, 5, 0, 1, 2, 4)
    ).reshape(B, S, H, CBS, D)
    v_sel = jnp.transpose(v_out.reshape(H, CBS, D, S), (3, 0, 1, 2)).reshape(
        B, S, H, CBS, D
    )
    return (k_sel, v_sel)


# speedup vs baseline: 1.0235x; 1.0235x over previous
import functools

import jax
import jax.numpy as jnp
from jax import lax
from jax.experimental import pallas as pl
from jax.experimental.pallas import tpu as pltpu
from jax.experimental.pallas import tpu_sc as plsc


def kernel(q, k_compressed, v_compressed):
    B, S, H, D = q.shape
    CBS = k_compressed.shape[3]
    HC = H * CBS
    DT, DI = D // 8, 8
    ST, SI = S // 128, 128

    ksrc = k_compressed.reshape(HC, D)
    vsrc = v_compressed.reshape(HC, D)

    info = plsc.get_sparse_core_info()
    NC = info.num_cores
    NW = NC * info.num_subcores
    BPW = HC // NW
    JOBS = BPW * DT
    RING = 4

    mesh = plsc.VectorSubcoreMesh(core_axis_name="c", subcore_axis_name="s")

    @functools.partial(
        pl.kernel,
        mesh=mesh,
        compiler_params=pltpu.CompilerParams(
            use_tc_tiling_on_sc=False, needs_layout_passes=False
        ),
        out_type=[jax.ShapeDtypeStruct((HC, DT, ST, DI, SI), jnp.float32)],
        scratch_types=[
            pltpu.VMEM((HC, D), jnp.float32),
            pltpu.VMEM((RING, ST, DI, SI), jnp.float32),
            pltpu.SemaphoreType.DMA,
        ],
    )
    def sc_write(ks_hbm, ko_hbm, ksv, buf, sem):
        wid = lax.axis_index("s") * NC + lax.axis_index("c")
        base = wid * BPW
        pltpu.sync_copy(ks_hbm, ksv)

        def job(j, carry):
            hc = base + j // DT
            dt = j % DT
            slot = j % RING

            @pl.when(j >= RING)
            def _():
                pltpu.make_async_copy(ko_hbm.at[0, 0], buf.at[0], sem).wait()

            hc_i = jnp.full((16,), hc, jnp.int32)
            splats = [
                plsc.load_gather(
                    ksv, [hc_i, jnp.full((16,), dt * DI + di, jnp.int32)]
                )
                for di in range(DI)
            ]

            def fill(st, c):
                for di in range(DI):
                    for i in range(SI // 16):
                        buf[slot, st, di, pl.ds(i * 16, 16)] = splats[di]
                return c

            lax.fori_loop(0, ST, fill, 0)
            pltpu.async_copy(buf.at[slot], ko_hbm.at[hc, dt], sem)
            return carry

        lax.fori_loop(0, JOBS, job, 0)
        for _ in range(RING):
            pltpu.make_async_copy(ko_hbm.at[0, 0], buf.at[0], sem).wait()

    (k_out,) = sc_write(ksrc)

    GC = 8
    grid = (HC // GC,)

    def tc_body(vs_ref, vo_ref):
        vo_ref[...] = jnp.broadcast_to(vs_ref[...][:, :, None], (GC, D, S))

    v_out = pl.pallas_call(
        tc_body,
        grid=grid,
        in_specs=[pl.BlockSpec((GC, D), lambda i: (i, 0))],
        out_specs=[pl.BlockSpec((GC, D, S), lambda i: (i, 0, 0))],
        out_shape=[jax.ShapeDtypeStruct((HC, D, S), jnp.float32)],
    )(vsrc)[0]

    k_sel = jnp.transpose(
        k_out.reshape(H, CBS, DT, ST, DI, SI), (3, 5, 0, 1, 2, 4)
    ).reshape(B, S, H, CBS, D)
    v_sel = jnp.transpose(v_out.reshape(H, CBS, D, S), (3, 0, 1, 2)).reshape(
        B, S, H, CBS, D
    )
    return (k_sel, v_sel)
